# baseline (speedup 1.0000x reference)
import functools

import jax
import jax.numpy as jnp
from jax import lax
from jax.experimental import pallas as pl
from jax.experimental.pallas import tpu as pltpu
from jax.experimental.pallas import tpu_sc as plsc

NC = 2
NS = 16
B = 128
RB = 512


def _cdiv(a, b):
    return (a + b - 1) // b


def _sc_mesh():
    return plsc.VectorSubcoreMesh(core_axis_name="c", subcore_axis_name="s")


def _spmm_call(xs, srcg2, dstd2, n_nodes):
    f = xs.shape[1]
    nchunk = srcg2.shape[0] // (NC * NS * B)
    acc_rows = 8 * NS * _cdiv(n_nodes + 1, 8 * NS)
    stripe = acc_rows // NS
    ostripe = 8 * (n_nodes // (8 * NS))
    olast = n_nodes - (NS - 1) * ostripe
    npair = nchunk // 2

    idx_t = pltpu.VMEM((B,), jnp.int32)

    @functools.partial(
        pl.kernel,
        out_type=jax.ShapeDtypeStruct((NC, n_nodes, f), jnp.float32),
        mesh=_sc_mesh(),
        scratch_types=[
            [idx_t] * 4,
            [idx_t] * 4,
            pltpu.VMEM((B, f), jnp.float32),
            pltpu.VMEM((B, f), jnp.float32),
            pltpu.VMEM_SHARED((acc_rows, f), jnp.float32),
            pltpu.SemaphoreType.DMA,
            pltpu.SemaphoreType.DMA,
            pltpu.SemaphoreType.DMA,
        ],
    )
    def k(xs_hbm, srcg_hbm, dstd_hbm, out_hbm, seta, setb,
          r0buf, r1buf, acc, gsem, ssem, isem):
        c = lax.axis_index("c")
        s = lax.axis_index("s")
        tid = c * NS + s
        base = tid * nchunk * B

        @pl.loop(0, B)
        def _(i):
            for j in range(f // 16):
                r0buf[pl.ds(i, 1), pl.ds(j * 16, 16)] = jnp.zeros(
                    (1, 16), jnp.float32)

        z0 = s * stripe
        done = 0
        while done < stripe:
            sz = min(B, stripe - done)
            pltpu.sync_copy(r0buf.at[pl.ds(0, sz)],
                            acc.at[pl.ds(z0 + done, sz)])
            done += sz

        def idx_load(pair, dst_set, sync=False):
            off = base + (2 * pair) * B
            pairs = [(srcg_hbm, dst_set[0]), (dstd_hbm, dst_set[1]),
                     (srcg_hbm, dst_set[2]), (dstd_hbm, dst_set[3])]
            for i, (src, dbuf) in enumerate(pairs):
                o = off + (B if i >= 2 else 0)
                if sync:
                    pltpu.sync_copy(src.at[pl.ds(o, B)], dbuf)
                else:
                    pltpu.async_copy(src.at[pl.ds(o, B)], dbuf, isem)

        def idx_drain(dst_set):
            for i in range(4):
                pltpu.make_async_copy(
                    srcg_hbm.at[pl.ds(base, B)], dst_set[i], isem).wait()

        def gwait(buf):
            pltpu.make_async_copy(
                xs_hbm.at[seta[0]], buf, gsem).wait()

        idx_load(0, seta, sync=True)
        pltpu.async_copy(xs_hbm.at[seta[0]], r0buf, gsem)
        pltpu.async_copy(xs_hbm.at[seta[2]], r1buf, gsem)
        plsc.subcore_barrier()

        def pair_body(m, cur, nxt, guard_next):
            if guard_next:
                @pl.when(m < npair - 1)
                def _():
                    idx_load(m + 1, nxt)
            else:
                idx_load(m + 1, nxt)
            gwait(r0buf)
            d0 = pltpu.async_copy(r0buf, acc.at[cur[1]], ssem, add=True)
            gwait(r1buf)
            d1 = pltpu.async_copy(r1buf, acc.at[cur[3]], ssem, add=True)
            d0.wait()
            d1.wait()

            def fire_next():
                idx_drain(nxt)
                pltpu.async_copy(xs_hbm.at[nxt[0]], r0buf, gsem)
                pltpu.async_copy(xs_hbm.at[nxt[2]], r1buf, gsem)

            if guard_next:
                @pl.when(m < npair - 1)
                def _():
                    fire_next()
            else:
                fire_next()

        @pl.loop(0, npair // 2)
        def _(mm):
            pair_body(2 * mm, seta, setb, guard_next=False)
            pair_body(2 * mm + 1, setb, seta, guard_next=True)

        plsc.subcore_barrier()

        @pl.when(s < NS - 1)
        def _():
            pltpu.sync_copy(acc.at[pl.ds(s * ostripe, ostripe)],
                            out_hbm.at[c, pl.ds(s * ostripe, ostripe)])

        @pl.when(s == NS - 1)
        def _():
            pltpu.sync_copy(acc.at[pl.ds((NS - 1) * ostripe, olast)],
                            out_hbm.at[c, pl.ds((NS - 1) * ostripe, olast)])

    return k(xs, srcg2, dstd2)


def _prep_body(degp_ref, x_ref, dis_ref, u0_ref, u1_ref, u2_ref, u3_ref):
    deg = degp_ref[0, :, 0:16] + degp_ref[1, :, 0:16]
    dis = jnp.where(deg > 0,
                    lax.rsqrt(jnp.maximum(deg, 1e-12)), 0.0)
    dis_ref[...] = dis
    d1 = dis[:, 0:1]
    u0_ref[...] = d1 * x_ref[0]
    u1_ref[...] = d1 * x_ref[1]
    u2_ref[...] = d1 * x_ref[2]
    u3_ref[...] = d1 * x_ref[3]


def _prep_call(degp, x_seq):
    t, n, f = x_seq.shape
    nb = _cdiv(n, RB)
    row = lambda r: (r, 0)
    out = jax.ShapeDtypeStruct((n, f), jnp.float32)
    return pl.pallas_call(
        _prep_body,
        grid=(nb,),
        in_specs=[
            pl.BlockSpec((2, RB, f), lambda r: (0, r, 0)),
            pl.BlockSpec((t, RB, f), lambda r: (0, r, 0)),
        ],
        out_specs=[pl.BlockSpec((RB, 16), row)] + [
            pl.BlockSpec((RB, f), row) for _ in range(t)],
        out_shape=[jax.ShapeDtypeStruct((n, 16), jnp.float32)] + [out] * t,
    )(degp, x_seq)


def _gates_body(x_ref, sx_ref, h_ref, sh_ref, dis_ref, w_ref, b_ref,
                z_ref, hr_ref, uhr_ref, txx_ref):
    d1 = dis_ref[:, 0:1]
    txx = -d1 * (sx_ref[0] + sx_ref[1])
    txh = -d1 * (sh_ref[0] + sh_ref[1])
    x = x_ref[...]
    h = h_ref[...]
    acc = jnp.dot(x, w_ref[0:128, :], preferred_element_type=jnp.float32)
    acc += jnp.dot(txx, w_ref[128:256, :], preferred_element_type=jnp.float32)
    acc += jnp.dot(h, w_ref[256:384, :], preferred_element_type=jnp.float32)
    acc += jnp.dot(txh, w_ref[384:512, :], preferred_element_type=jnp.float32)
    zr = jax.nn.sigmoid(acc + b_ref[...])
    z = zr[:, 0:128]
    r = zr[:, 128:256]
    hr = h * r
    z_ref[...] = z
    hr_ref[...] = hr
    uhr_ref[...] = d1 * hr
    txx_ref[...] = txx


def _gates_call(x, sx, h, sh, dis16, wzr, bzr):
    n, f = x.shape
    nb = _cdiv(n, RB)
    row = lambda r: (r, 0)
    nf = jax.ShapeDtypeStruct((n, f), jnp.float32)
    return pl.pallas_call(
        _gates_body,
        grid=(nb,),
        in_specs=[
            pl.BlockSpec((RB, f), row),
            pl.BlockSpec((2, RB, f), lambda r: (0, r, 0)),
            pl.BlockSpec((RB, f), row),
            pl.BlockSpec((2, RB, f), lambda r: (0, r, 0)),
            pl.BlockSpec((RB, 16), row),
            pl.BlockSpec((512, 256), lambda r: (0, 0)),
            pl.BlockSpec((1, 256), lambda r: (0, 0)),
        ],
        out_specs=[pl.BlockSpec((RB, f), row)] * 4,
        out_shape=[nf, nf, nf, nf],
    )(x, sx, h, sh, dis16, wzr, bzr)


def _update_body(x_ref, txx_ref, hr_ref, shr_ref, z_ref, h_ref, dis_ref,
                 w_ref, b_ref, hn_ref, uh_ref):
    d1 = dis_ref[:, 0:1]
    txhr = -d1 * (shr_ref[0] + shr_ref[1])
    acc = jnp.dot(x_ref[...], w_ref[0:128, :],
                  preferred_element_type=jnp.float32)
    acc += jnp.dot(txx_ref[...], w_ref[128:256, :],
                   preferred_element_type=jnp.float32)
    acc += jnp.dot(hr_ref[...], w_ref[256:384, :],
                   preferred_element_type=jnp.float32)
    acc += jnp.dot(txhr, w_ref[384:512, :],
                   preferred_element_type=jnp.float32)
    ht = jnp.tanh(acc + b_ref[...])
    z = z_ref[...]
    hn = z * h_ref[...] + (1.0 - z) * ht
    hn_ref[...] = hn
    uh_ref[...] = d1 * hn


def _update_call(x, txx, hr, shr, z, h, dis16, wh, bh):
    n, f = x.shape
    nb = _cdiv(n, RB)
    row = lambda r: (r, 0)
    nf = jax.ShapeDtypeStruct((n, f), jnp.float32)
    return pl.pallas_call(
        _update_body,
        grid=(nb,),
        in_specs=[
            pl.BlockSpec((RB, f), row),
            pl.BlockSpec((RB, f), row),
            pl.BlockSpec((RB, f), row),
            pl.BlockSpec((2, RB, f), lambda r: (0, r, 0)),
            pl.BlockSpec((RB, f), row),
            pl.BlockSpec((RB, f), row),
            pl.BlockSpec((RB, 16), row),
            pl.BlockSpec((512, 128), lambda r: (0, 0)),
            pl.BlockSpec((1, 128), lambda r: (0, 0)),
        ],
        out_specs=[pl.BlockSpec((RB, f), row)] * 2,
        out_shape=[nf, nf],
    )(x, txx, hr, shr, z, h, dis16, wh, bh)


def _final_body(h_ref, w_ref, b_ref, o_ref):
    o_ref[...] = jnp.dot(h_ref[...], w_ref[...],
                         preferred_element_type=jnp.float32) + b_ref[...]


def _final_call(h, wlin, blin):
    n, f = h.shape
    fo = wlin.shape[1]
    nb = _cdiv(n, RB)
    return pl.pallas_call(
        _final_body,
        grid=(nb,),
        in_specs=[
            pl.BlockSpec((RB, f), lambda r: (r, 0)),
            pl.BlockSpec((f, fo), lambda r: (0, 0)),
            pl.BlockSpec((1, fo), lambda r: (0, 0)),
        ],
        out_specs=pl.BlockSpec((RB, fo), lambda r: (r, 0)),
        out_shape=jax.ShapeDtypeStruct((n, fo), jnp.float32),
    )(h, wlin, blin.reshape(1, fo))


def kernel(X_seq, edge_index, Wxz, bxz, Whz, bhz, Wxr, bxr, Whr, bhr,
           Wxh, bxh, Whh, bhh, Wlin, blin):
    t_steps, n, f = X_seq.shape
    e = edge_index.shape[1]

    src = edge_index[0].astype(jnp.int32)
    dst = edge_index[1].astype(jnp.int32)
    self_loop = src == dst
    srcg = jnp.where(self_loop, 0, src)
    dstd = jnp.where(self_loop, n, dst)
    srcd = jnp.where(self_loop, n, src)

    nchunk = 8 * _cdiv(e, NC * NS * B * 8)
    e_pad = NC * NS * B * nchunk
    srcg2 = jnp.pad(srcg, (0, e_pad - e))
    srcd2 = jnp.pad(srcd, (0, e_pad - e), constant_values=n)
    dstd2 = jnp.pad(dstd, (0, e_pad - e), constant_values=n)

    wzr = jnp.concatenate([
        jnp.concatenate([Wxz[0], Wxr[0]], axis=1),
        jnp.concatenate([Wxz[1], Wxr[1]], axis=1),
        jnp.concatenate([Whz[0], Whr[0]], axis=1),
        jnp.concatenate([Whz[1], Whr[1]], axis=1),
    ], axis=0)
    bzr = jnp.concatenate([bxz + bhz, bxr + bhr]).reshape(1, 256)
    wh = jnp.concatenate([Wxh[0], Wxh[1], Whh[0], Whh[1]], axis=0)
    bh = (bxh + bhh).reshape(1, 128)

    degp = _spmm_call(jnp.ones((n, f), jnp.float32), srcg2, srcd2, n)
    prep = _prep_call(degp, X_seq)
    dis16, us = prep[0], prep[1:]

    sx = [_spmm_call(u, srcg2, dstd2, n) for u in us]

    h = jnp.zeros((n, f), jnp.float32)
    sh = jnp.zeros((NC, n, f), jnp.float32)
    for t in range(t_steps):
        z, hr, uhr, txx = _gates_call(X_seq[t], sx[t], h, sh, dis16, wzr, bzr)
        shr = _spmm_call(uhr, srcg2, dstd2, n)
        h, uh = _update_call(X_seq[t], txx, hr, shr, z, h, dis16, wh, bh)
        if t < t_steps - 1:
            sh = _spmm_call(uh, srcg2, dstd2, n)

    return _final_call(h, Wlin, blin)


# speedup vs baseline: 1.0174x; 1.0174x over previous
import functools

import jax
import jax.numpy as jnp
from jax import lax
from jax.experimental import pallas as pl
from jax.experimental.pallas import tpu as pltpu
from jax.experimental.pallas import tpu_sc as plsc

NC = 2
NS = 16
B = 128
RB = 512


def _cdiv(a, b):
    return (a + b - 1) // b


def _sc_mesh():
    return plsc.VectorSubcoreMesh(core_axis_name="c", subcore_axis_name="s")


def _spmm_call(xs, idx3, n_nodes):
    f = xs.shape[1]
    nchunk = idx3.shape[0] // (NC * NS)
    acc_rows = 8 * NS * _cdiv(n_nodes + 1, 8 * NS)
    stripe = acc_rows // NS
    ostripe = 8 * (n_nodes // (8 * NS))
    olast = n_nodes - (NS - 1) * ostripe

    @functools.partial(
        pl.kernel,
        out_type=jax.ShapeDtypeStruct((NC, n_nodes, f), jnp.float32),
        mesh=_sc_mesh(),
        scratch_types=[
            pltpu.VMEM((2, B), jnp.int32),
            pltpu.VMEM((2, B), jnp.int32),
            pltpu.VMEM((B, f), jnp.float32),
            pltpu.VMEM((B, f), jnp.float32),
            pltpu.VMEM_SHARED((acc_rows, f), jnp.float32),
            pltpu.SemaphoreType.DMA,
            pltpu.SemaphoreType.DMA,
        ],
    )
    def k(xs_hbm, idx_hbm, out_hbm, i0, i1, r0buf, r1buf, acc, gsem, isem):
        c = lax.axis_index("c")
        s = lax.axis_index("s")
        tid = c * NS + s
        base = tid * nchunk

        @pl.loop(0, B)
        def _(i):
            for j in range(f // 16):
                r0buf[pl.ds(i, 1), pl.ds(j * 16, 16)] = jnp.zeros(
                    (1, 16), jnp.float32)

        z0 = s * stripe
        done = 0
        while done < stripe:
            sz = min(B, stripe - done)
            pltpu.sync_copy(r0buf.at[pl.ds(0, sz)],
                            acc.at[pl.ds(z0 + done, sz)])
            done += sz

        def idx_fire(k_, ibuf):
            pltpu.async_copy(idx_hbm.at[base + k_], ibuf, isem)

        def idx_wait(ibuf):
            pltpu.make_async_copy(idx_hbm.at[base], ibuf, isem).wait()

        def gwait(buf):
            pltpu.make_async_copy(xs_hbm.at[i0.at[0]], buf, gsem).wait()

        pltpu.sync_copy(idx_hbm.at[base], i0)
        pltpu.async_copy(xs_hbm.at[i0.at[0]], r0buf, gsem)
        idx_fire(1, i1)
        plsc.subcore_barrier()

        def chunk_body(k_, icur, inxt, rcur, rnxt, last):
            gwait(rcur)
            if not last:
                idx_wait(inxt)
                pltpu.async_copy(xs_hbm.at[inxt.at[0]], rnxt, gsem)
            pltpu.sync_copy(rcur, acc.at[icur.at[1]], add=True)
            if not last:
                @pl.when(k_ + 2 < nchunk)
                def _():
                    idx_fire(k_ + 2, icur)

        @pl.loop(0, nchunk // 2 - 1)
        def _(mm):
            chunk_body(2 * mm, i0, i1, r0buf, r1buf, last=False)
            chunk_body(2 * mm + 1, i1, i0, r1buf, r0buf, last=False)

        chunk_body(nchunk - 2, i0, i1, r0buf, r1buf, last=False)
        chunk_body(nchunk - 1, i1, i0, r1buf, r0buf, last=True)

        plsc.subcore_barrier()

        @pl.when(s < NS - 1)
        def _():
            pltpu.sync_copy(acc.at[pl.ds(s * ostripe, ostripe)],
                            out_hbm.at[c, pl.ds(s * ostripe, ostripe)])

        @pl.when(s == NS - 1)
        def _():
            pltpu.sync_copy(acc.at[pl.ds((NS - 1) * ostripe, olast)],
                            out_hbm.at[c, pl.ds((NS - 1) * ostripe, olast)])

    return k(xs, idx3)


def _prep_body(degp_ref, x_ref, dis_ref, u0_ref, u1_ref, u2_ref, u3_ref):
    deg = degp_ref[0, :, 0:16] + degp_ref[1, :, 0:16]
    dis = jnp.where(deg > 0,
                    lax.rsqrt(jnp.maximum(deg, 1e-12)), 0.0)
    dis_ref[...] = dis
    d1 = dis[:, 0:1]
    u0_ref[...] = d1 * x_ref[0]
    u1_ref[...] = d1 * x_ref[1]
    u2_ref[...] = d1 * x_ref[2]
    u3_ref[...] = d1 * x_ref[3]


def _prep_call(degp, x_seq):
    t, n, f = x_seq.shape
    nb = _cdiv(n, RB)
    row = lambda r: (r, 0)
    out = jax.ShapeDtypeStruct((n, f), jnp.float32)
    return pl.pallas_call(
        _prep_body,
        grid=(nb,),
        in_specs=[
            pl.BlockSpec((2, RB, f), lambda r: (0, r, 0)),
            pl.BlockSpec((t, RB, f), lambda r: (0, r, 0)),
        ],
        out_specs=[pl.BlockSpec((RB, 16), row)] + [
            pl.BlockSpec((RB, f), row) for _ in range(t)],
        out_shape=[jax.ShapeDtypeStruct((n, 16), jnp.float32)] + [out] * t,
    )(degp, x_seq)


def _gates_body(x_ref, sx_ref, h_ref, sh_ref, dis_ref, w_ref, b_ref,
                z_ref, hr_ref, uhr_ref, txx_ref):
    d1 = dis_ref[:, 0:1]
    txx = -d1 * (sx_ref[0] + sx_ref[1])
    txh = -d1 * (sh_ref[0] + sh_ref[1])
    x = x_ref[...]
    h = h_ref[...]
    acc = jnp.dot(x, w_ref[0:128, :], preferred_element_type=jnp.float32)
    acc += jnp.dot(txx, w_ref[128:256, :], preferred_element_type=jnp.float32)
    acc += jnp.dot(h, w_ref[256:384, :], preferred_element_type=jnp.float32)
    acc += jnp.dot(txh, w_ref[384:512, :], preferred_element_type=jnp.float32)
    zr = jax.nn.sigmoid(acc + b_ref[...])
    z = zr[:, 0:128]
    r = zr[:, 128:256]
    hr = h * r
    z_ref[...] = z
    hr_ref[...] = hr
    uhr_ref[...] = d1 * hr
    txx_ref[...] = txx


def _gates_call(x, sx, h, sh, dis16, wzr, bzr):
    n, f = x.shape
    nb = _cdiv(n, RB)
    row = lambda r: (r, 0)
    nf = jax.ShapeDtypeStruct((n, f), jnp.float32)
    return pl.pallas_call(
        _gates_body,
        grid=(nb,),
        in_specs=[
            pl.BlockSpec((RB, f), row),
            pl.BlockSpec((2, RB, f), lambda r: (0, r, 0)),
            pl.BlockSpec((RB, f), row),
            pl.BlockSpec((2, RB, f), lambda r: (0, r, 0)),
            pl.BlockSpec((RB, 16), row),
            pl.BlockSpec((512, 256), lambda r: (0, 0)),
            pl.BlockSpec((1, 256), lambda r: (0, 0)),
        ],
        out_specs=[pl.BlockSpec((RB, f), row)] * 4,
        out_shape=[nf, nf, nf, nf],
    )(x, sx, h, sh, dis16, wzr, bzr)


def _update_body(x_ref, txx_ref, hr_ref, shr_ref, z_ref, h_ref, dis_ref,
                 w_ref, b_ref, hn_ref, uh_ref):
    d1 = dis_ref[:, 0:1]
    txhr = -d1 * (shr_ref[0] + shr_ref[1])
    acc = jnp.dot(x_ref[...], w_ref[0:128, :],
                  preferred_element_type=jnp.float32)
    acc += jnp.dot(txx_ref[...], w_ref[128:256, :],
                   preferred_element_type=jnp.float32)
    acc += jnp.dot(hr_ref[...], w_ref[256:384, :],
                   preferred_element_type=jnp.float32)
    acc += jnp.dot(txhr, w_ref[384:512, :],
                   preferred_element_type=jnp.float32)
    ht = jnp.tanh(acc + b_ref[...])
    z = z_ref[...]
    hn = z * h_ref[...] + (1.0 - z) * ht
    hn_ref[...] = hn
    uh_ref[...] = d1 * hn


def _update_call(x, txx, hr, shr, z, h, dis16, wh, bh):
    n, f = x.shape
    nb = _cdiv(n, RB)
    row = lambda r: (r, 0)
    nf = jax.ShapeDtypeStruct((n, f), jnp.float32)
    return pl.pallas_call(
        _update_body,
        grid=(nb,),
        in_specs=[
            pl.BlockSpec((RB, f), row),
            pl.BlockSpec((RB, f), row),
            pl.BlockSpec((RB, f), row),
            pl.BlockSpec((2, RB, f), lambda r: (0, r, 0)),
            pl.BlockSpec((RB, f), row),
            pl.BlockSpec((RB, f), row),
            pl.BlockSpec((RB, 16), row),
            pl.BlockSpec((512, 128), lambda r: (0, 0)),
            pl.BlockSpec((1, 128), lambda r: (0, 0)),
        ],
        out_specs=[pl.BlockSpec((RB, f), row)] * 2,
        out_shape=[nf, nf],
    )(x, txx, hr, shr, z, h, dis16, wh, bh)


def _final_body(h_ref, w_ref, b_ref, o_ref):
    o_ref[...] = jnp.dot(h_ref[...], w_ref[...],
                         preferred_element_type=jnp.float32) + b_ref[...]


def _final_call(h, wlin, blin):
    n, f = h.shape
    fo = wlin.shape[1]
    nb = _cdiv(n, RB)
    return pl.pallas_call(
        _final_body,
        grid=(nb,),
        in_specs=[
            pl.BlockSpec((RB, f), lambda r: (r, 0)),
            pl.BlockSpec((f, fo), lambda r: (0, 0)),
            pl.BlockSpec((1, fo), lambda r: (0, 0)),
        ],
        out_specs=pl.BlockSpec((RB, fo), lambda r: (r, 0)),
        out_shape=jax.ShapeDtypeStruct((n, fo), jnp.float32),
    )(h, wlin, blin.reshape(1, fo))


def kernel(X_seq, edge_index, Wxz, bxz, Whz, bhz, Wxr, bxr, Whr, bhr,
           Wxh, bxh, Whh, bhh, Wlin, blin):
    t_steps, n, f = X_seq.shape
    e = edge_index.shape[1]

    src = edge_index[0].astype(jnp.int32)
    dst = edge_index[1].astype(jnp.int32)
    self_loop = src == dst
    srcg = jnp.where(self_loop, 0, src)
    dstd = jnp.where(self_loop, n, dst)
    srcd = jnp.where(self_loop, n, src)

    nchunk = 8 * _cdiv(e, NC * NS * B * 8)
    e_pad = NC * NS * B * nchunk
    srcg2 = jnp.pad(srcg, (0, e_pad - e)).reshape(-1, B)
    srcd2 = jnp.pad(srcd, (0, e_pad - e),
                    constant_values=n).reshape(-1, B)
    dstd2 = jnp.pad(dstd, (0, e_pad - e),
                    constant_values=n).reshape(-1, B)
    idx_main = jnp.stack([srcg2, dstd2], axis=1)
    idx_deg = jnp.stack([srcg2, srcd2], axis=1)

    wzr = jnp.concatenate([
        jnp.concatenate([Wxz[0], Wxr[0]], axis=1),
        jnp.concatenate([Wxz[1], Wxr[1]], axis=1),
        jnp.concatenate([Whz[0], Whr[0]], axis=1),
        jnp.concatenate([Whz[1], Whr[1]], axis=1),
    ], axis=0)
    bzr = jnp.concatenate([bxz + bhz, bxr + bhr]).reshape(1, 256)
    wh = jnp.concatenate([Wxh[0], Wxh[1], Whh[0], Whh[1]], axis=0)
    bh = (bxh + bhh).reshape(1, 128)

    degp = _spmm_call(jnp.ones((n, f), jnp.float32), idx_deg, n)
    prep = _prep_call(degp, X_seq)
    dis16, us = prep[0], prep[1:]

    sx = [_spmm_call(u, idx_main, n) for u in us]

    h = jnp.zeros((n, f), jnp.float32)
    sh = jnp.zeros((NC, n, f), jnp.float32)
    for t in range(t_steps):
        z, hr, uhr, txx = _gates_call(X_seq[t], sx[t], h, sh, dis16, wzr, bzr)
        shr = _spmm_call(uhr, idx_main, n)
        h, uh = _update_call(X_seq[t], txx, hr, shr, z, h, dis16, wh, bh)
        if t < t_steps - 1:
            sh = _spmm_call(uh, idx_main, n)

    return _final_call(h, Wlin, blin)
